# Initial kernel scaffold; baseline (speedup 1.0000x reference)
import jax
import jax.numpy as jnp
from jax.experimental import pallas as pl


def kernel(gt_bboxes, gt_labels, pred_bboxes, pred_labels):
    raise NotImplementedError("write your pallas kernel here")


# speedup vs baseline: 8.5815x; 8.5815x over previous
import functools

import jax
import jax.numpy as jnp
from jax.experimental import pallas as pl
from jax.experimental.pallas import tpu as pltpu

RATIO_POS = 3
NUM_CLASSES = 21
B, A = 32, 8732
MININT = -2147483648


def _ssd_kernel(gt_ref, pr_ref, lab_ref, logit_ref, reg_ref, cls_ref,
                conf_s, key_s, acc_s):
    b = pl.program_id(0)

    @pl.when(b == 0)
    def _init():
        acc_s[0] = 0.0
        acc_s[1] = 0.0
        acc_s[2] = 0.0

    gt = gt_ref[0]
    pr = pr_ref[0]
    lab = lab_ref[0]
    x = logit_ref[0]

    pos = lab > 0
    posf = pos.astype(jnp.float32)

    d = pr - gt
    ad = jnp.abs(d)
    sl1 = jnp.where(ad < 1.0, 0.5 * d * d, ad - 0.5)
    acc_s[0] += jnp.sum(jnp.sum(sl1, axis=0, keepdims=True) * posf)

    acc_s[1] += jnp.sum((jnp.sum(gt, axis=0, keepdims=True) > 0)
                        .astype(jnp.float32))

    m = jnp.max(x, axis=0, keepdims=True)
    lse = m + jnp.log(jnp.sum(jnp.exp(x - m), axis=0, keepdims=True))
    cls_iota = jax.lax.broadcasted_iota(jnp.int32, x.shape, 0)
    chosen = jnp.sum(jnp.where(cls_iota == lab, x, 0.0), axis=0,
                     keepdims=True)
    conf = jnp.maximum(lse - chosen, 0.0)

    acc_s[2] += jnp.sum(conf * posf)

    aidx = jax.lax.broadcasted_iota(jnp.int32, (1, A), 1)
    confbits = jax.lax.bitcast_convert_type(conf, jnp.int32)
    key = jnp.where(pos, -(aidx + 1), confbits)

    conf_s[pl.ds(b, 1), :] = conf
    key_s[pl.ds(b, 1), :] = key

    @pl.when(b == pl.num_programs(0) - 1)
    def _mine():
        keys = key_s[:, :]
        confs = conf_s[:, :]
        t = RATIO_POS * jnp.sum((keys < 0).astype(jnp.int32), axis=1,
                                keepdims=True)

        def step(i, p):
            bit = jax.lax.shift_left(jnp.int32(1), jnp.int32(31) - i)
            cand = p | bit
            cnt = jnp.sum((keys >= (cand ^ MININT)).astype(jnp.int32),
                          axis=1, keepdims=True)
            return jnp.where(cnt >= t, cand, p)

        p = jax.lax.fori_loop(0, 32, step, jnp.zeros((B, 1), jnp.int32))
        theta = p ^ MININT

        gt_m = keys > theta
        eq_m = keys == theta
        c_gt = jnp.sum(gt_m.astype(jnp.float32), axis=1, keepdims=True)
        c_eq = jnp.sum(eq_m.astype(jnp.float32), axis=1, keepdims=True)
        s_gt = jnp.sum(jnp.where(gt_m, confs, 0.0), axis=1, keepdims=True)
        s_eq = jnp.sum(jnp.where(eq_m, confs, 0.0), axis=1, keepdims=True)
        tie = jnp.where(c_eq > 0.0,
                        (t.astype(jnp.float32) - c_gt) * s_eq
                        / jnp.where(c_eq > 0.0, c_eq, 1.0),
                        0.0)
        s_bg = jnp.sum(s_gt + tie)

        num_pos = acc_s[1]
        reg_ref[0] = acc_s[0] / num_pos
        cls_ref[0] = (acc_s[2] + s_bg) / num_pos


@functools.partial(jax.jit, static_argnames=("interpret",))
def kernel(gt_bboxes, gt_labels, pred_bboxes, pred_labels, interpret=False):
    gt_t = jnp.transpose(gt_bboxes, (0, 2, 1))
    pr_t = jnp.transpose(pred_bboxes, (0, 2, 1))
    lab3 = gt_labels.reshape(B, 1, A).astype(jnp.int32)
    logit_t = jnp.transpose(pred_labels, (0, 2, 1))

    reg, cls = pl.pallas_call(
        _ssd_kernel,
        grid=(B,),
        in_specs=[
            pl.BlockSpec((1, 4, A), lambda b: (b, 0, 0)),
            pl.BlockSpec((1, 4, A), lambda b: (b, 0, 0)),
            pl.BlockSpec((1, 1, A), lambda b: (b, 0, 0)),
            pl.BlockSpec((1, NUM_CLASSES, A), lambda b: (b, 0, 0)),
        ],
        out_specs=[
            pl.BlockSpec(memory_space=pltpu.SMEM),
            pl.BlockSpec(memory_space=pltpu.SMEM),
        ],
        out_shape=[
            jax.ShapeDtypeStruct((1,), jnp.float32),
            jax.ShapeDtypeStruct((1,), jnp.float32),
        ],
        scratch_shapes=[
            pltpu.VMEM((B, A), jnp.float32),
            pltpu.VMEM((B, A), jnp.int32),
            pltpu.SMEM((4,), jnp.float32),
        ],
        interpret=interpret,
    )(gt_t, pr_t, lab3, logit_t)
    return (reg[0], cls[0])
